# baseline (speedup 1.0000x reference)
import jax
import jax.numpy as jnp
import numpy as np
from jax.experimental import pallas as pl
from jax.experimental.pallas import tpu as pltpu

_B, _C, _R, _A, _NANG, _HID = 8, 4, 2048, 32, 8, 32
_NAA = 20
_NP = 32
_MAXCHI = 5
_RB = 512
_N = _B * _C * _A
_NC = 256
_NFEA = {'GLN': 3, 'VAL': 1, 'ASN': 2, 'THR': 1, 'ASP': 2, 'PHE': 2, 'LEU': 2,
         'SER': 1, 'CYS': 1, 'ILE': 1, 'TRP': 2, 'ARG': 5, 'LYS': 4, 'TYR': 2,
         'GLU': 3, 'MET': 3, 'HIS': 2}
_RESI = ['ALA', 'ARG', 'ASN', 'ASP', 'CYS', 'GLN', 'GLU', 'GLY', 'HIS', 'ILE',
         'LEU', 'LYS', 'MET', 'PHE', 'PRO', 'SER', 'THR', 'TRP', 'TYR', 'VAL']
_NCHI = [_NFEA.get(_RESI[i], 0) for i in range(_NAA)]


def _fused_kernel(x_ref, w1_ref, b1_ref, w2_ref, b2_ref, wpk_ref,
                  bb_ref, rot_ref):
    i = pl.program_id(0)
    bb_ref[...] = jnp.zeros_like(bb_ref)
    rot_ref[...] = jnp.zeros_like(rot_ref)

    @pl.when(i == 0)
    def _():
        s = 1.0 + jnp.tanh(wpk_ref[...])
        b2 = b2_ref[...]
        w2 = w2_ref[...]
        b1 = b1_ref[...]
        nbc = _NC // _A
        for c in range(_N // _NC):
            X = x_ref[:, :, c * _NC:(c + 1) * _NC]
            acc = b1[:, :, None]
            for f in range(_NANG):
                wf = w1_ref[:, f, :]
                acc = acc + X[:, f:f + 1, :] * wf[:, :, None]
            y = jnp.tanh(acc) * w2[:, :, None]
            bb_raw = jnp.sum(y[:, 0:_HID], axis=1) + b2[:, 0:1]
            om_raw = jnp.sum(y[:, _HID:2 * _HID], axis=1) + b2[:, 1:2]
            sc_raw = jnp.sum(y[:, 2 * _HID:], axis=1) + b2[:, 2:3]
            bb_p = jnp.minimum(bb_raw * s[:, 0:1], 5.0)
            om_p = om_raw * s[:, 1:2]
            sc_p = jnp.minimum(sc_raw * s[:, 2:3], 5.0)
            score = jnp.clip(-(bb_p + om_p + sc_p), 0.0, 5.0)
            scores = score.reshape(_NP, nbc, _A).transpose(1, 0, 2)
            nb = nbc // _C
            bb_ref[c * nb:(c + 1) * nb, :, 0:_NP, :] = (
                scores.reshape(nb, _C, _NP, _A))


def _pack_params(kde_params, weight_bb, weight_omega, weight_sc):
    sc_keys = [i for i in range(_NAA) if str(i) in kde_params['sc']]
    offs, cursor = {}, 0
    for i in sc_keys:
        offs[i] = cursor
        cursor += _NCHI[i]
    pos = {i: p for p, i in enumerate(sc_keys)}

    w1_leaves = ([kde_params['bb'][str(i)]['W1'] for i in range(_NAA)]
                 + [kde_params['omega'][str(i)]['W1'] for i in range(_NAA)]
                 + [kde_params['sc'][str(i)]['W1'] for i in sc_keys]
                 + [jnp.zeros((1, _HID), jnp.float32)])

    def leaves(name):
        return [kde_params[grp][str(i)][name]
                for grp in ('bb', 'omega', 'sc') for i in range(_NAA)
                if str(i) in kde_params[grp]]

    b1_cat = jnp.concatenate(leaves('b1'))
    w2_cat = jnp.concatenate(leaves('W2'), axis=0)
    rows = jnp.concatenate(
        w1_leaves + [b1_cat.reshape(-1, _HID), w2_cat.reshape(-1, _HID)],
        axis=0)

    B_OM, B_SC = 40, 60
    ZR = 97
    B_B1, B_W2 = 98, 98 + 57

    def sc_row(g, k):
        return (B_SC + offs[g] + k
                if g in offs and k < _NCHI[g] else ZR)

    idx = []
    for g in range(_NP):
        for f in range(_NANG):
            if g < _NAA:
                idx += [2 * g + f if f < 2 else ZR,
                        B_OM + g if f == 2 else ZR,
                        sc_row(g, f - 3) if f >= 3 else ZR]
            else:
                idx += [ZR, ZR, ZR]
    for base in (B_B1, B_W2):
        for g in range(_NP):
            if g < _NAA:
                idx += [base + g, base + 20 + g,
                        base + 40 + pos[g] if g in pos else ZR]
            else:
                idx += [ZR, ZR, ZR]
    assert rows.shape[0] == 212
    picked = jnp.take(rows, jnp.asarray(idx, jnp.int32), axis=0)
    nw1 = _NP * _NANG * 3
    w1 = picked[:nw1].reshape(_NP, _NANG, 3 * _HID)
    b1 = picked[nw1:nw1 + _NP * 3].reshape(_NP, 3 * _HID)
    w2 = picked[nw1 + _NP * 3:].reshape(_NP, 3 * _HID)

    svec = jnp.concatenate(
        [kde_params[grp][str(i)]['b2']
         for grp in ('bb', 'omega', 'sc') for i in range(_NAA)
         if str(i) in kde_params[grp]]
        + [weight_bb, weight_omega, weight_sc,
           jnp.zeros((1,), jnp.float32)])
    SZ = 79
    sidx = []
    for g in range(_NP):
        sidx += ([g, 20 + g, 40 + pos[g] if g in pos else SZ]
                 if g < _NAA else [SZ, SZ, SZ])
    for g in range(_NP):
        sidx += [57, 58, 59 + g if g < _NAA else SZ]
    spicked = jnp.take(svec, jnp.asarray(sidx, jnp.int32))
    b2 = spicked[:_NP * 3].reshape(_NP, 3)
    wpk = spicked[_NP * 3:].reshape(_NP, 3)
    return w1, b1, w2, b2, wpk


def kernel(atom_description, angles, alternatives, weight_omega, weight_bb,
           weight_sc, kde_params):
    naltern = alternatives.shape[-1]
    assert naltern == _A and angles.shape == (_B, _C, _R, _A, _NANG)

    slab = jnp.transpose(angles[:, :, :_NP], (2, 4, 0, 1, 3))
    slab = slab.reshape(_NP, _NANG, _N)

    w1, b1, w2, b2, wpk = _pack_params(
        kde_params, weight_bb, weight_omega, weight_sc)

    full = lambda a: pl.BlockSpec(a.shape, lambda i: (0,) * a.ndim)
    ins = (slab, w1, b1, w2, b2, wpk)
    out_spec = pl.BlockSpec((_B, _C, _RB, _A), lambda i: (0, 0, i, 0))
    out_sd = jax.ShapeDtypeStruct((_B, _C, _R, _A), jnp.float32)
    bb_score, rot = pl.pallas_call(
        _fused_kernel,
        grid=(_R // _RB,),
        in_specs=[full(a) for a in ins],
        out_specs=(out_spec, out_spec),
        out_shape=(out_sd, out_sd),
        compiler_params=pltpu.CompilerParams(
            dimension_semantics=("arbitrary",)),
    )(*ins)
    return (bb_score, rot)


# speedup vs baseline: 1.0051x; 1.0051x over previous
import jax
import jax.numpy as jnp
import numpy as np
from jax.experimental import pallas as pl
from jax.experimental.pallas import tpu as pltpu

_B, _C, _R, _A, _NANG, _HID = 8, 4, 2048, 32, 8, 32
_NAA = 20
_NP = 32
_MAXCHI = 5
_RB = 256
_N = _B * _C * _A
_NC = 256
_NFEA = {'GLN': 3, 'VAL': 1, 'ASN': 2, 'THR': 1, 'ASP': 2, 'PHE': 2, 'LEU': 2,
         'SER': 1, 'CYS': 1, 'ILE': 1, 'TRP': 2, 'ARG': 5, 'LYS': 4, 'TYR': 2,
         'GLU': 3, 'MET': 3, 'HIS': 2}
_RESI = ['ALA', 'ARG', 'ASN', 'ASP', 'CYS', 'GLN', 'GLU', 'GLY', 'HIS', 'ILE',
         'LEU', 'LYS', 'MET', 'PHE', 'PRO', 'SER', 'THR', 'TRP', 'TYR', 'VAL']
_NCHI = [_NFEA.get(_RESI[i], 0) for i in range(_NAA)]


def _fused_kernel(x_ref, w1_ref, b1_ref, w2_ref, b2_ref, wpk_ref,
                  bb_ref, rot_ref):
    i = pl.program_id(0)
    bb_ref[...] = jnp.zeros_like(bb_ref)
    rot_ref[...] = jnp.zeros_like(rot_ref)

    @pl.when(i == 0)
    def _():
        s = 1.0 + jnp.tanh(wpk_ref[...])
        b2 = b2_ref[...]
        w2 = w2_ref[...]
        b1 = b1_ref[...]
        nbc = _NC // _A
        for c in range(_N // _NC):
            X = x_ref[:, :, c * _NC:(c + 1) * _NC]
            acc = b1[:, :, None]
            for f in range(_NANG):
                wf = w1_ref[:, f, :]
                acc = acc + X[:, f:f + 1, :] * wf[:, :, None]
            y = jnp.tanh(acc) * w2[:, :, None]
            bb_raw = jnp.sum(y[:, 0:_HID], axis=1) + b2[:, 0:1]
            om_raw = jnp.sum(y[:, _HID:2 * _HID], axis=1) + b2[:, 1:2]
            sc_raw = jnp.sum(y[:, 2 * _HID:], axis=1) + b2[:, 2:3]
            bb_p = jnp.minimum(bb_raw * s[:, 0:1], 5.0)
            om_p = om_raw * s[:, 1:2]
            sc_p = jnp.minimum(sc_raw * s[:, 2:3], 5.0)
            score = jnp.clip(-(bb_p + om_p + sc_p), 0.0, 5.0)
            scores = score.reshape(_NP, nbc, _A).transpose(1, 0, 2)
            nb = nbc // _C
            bb_ref[c * nb:(c + 1) * nb, :, 0:_NP, :] = (
                scores.reshape(nb, _C, _NP, _A))


def _pack_params(kde_params, weight_bb, weight_omega, weight_sc):
    sc_keys = [i for i in range(_NAA) if str(i) in kde_params['sc']]
    offs, cursor = {}, 0
    for i in sc_keys:
        offs[i] = cursor
        cursor += _NCHI[i]
    pos = {i: p for p, i in enumerate(sc_keys)}

    w1_leaves = ([kde_params['bb'][str(i)]['W1'] for i in range(_NAA)]
                 + [kde_params['omega'][str(i)]['W1'] for i in range(_NAA)]
                 + [kde_params['sc'][str(i)]['W1'] for i in sc_keys]
                 + [jnp.zeros((1, _HID), jnp.float32)])

    def leaves(name):
        return [kde_params[grp][str(i)][name]
                for grp in ('bb', 'omega', 'sc') for i in range(_NAA)
                if str(i) in kde_params[grp]]

    b1_cat = jnp.concatenate(leaves('b1'))
    w2_cat = jnp.concatenate(leaves('W2'), axis=0)
    rows = jnp.concatenate(
        w1_leaves + [b1_cat.reshape(-1, _HID), w2_cat.reshape(-1, _HID)],
        axis=0)

    B_OM, B_SC = 40, 60
    ZR = 97
    B_B1, B_W2 = 98, 98 + 57

    def sc_row(g, k):
        return (B_SC + offs[g] + k
                if g in offs and k < _NCHI[g] else ZR)

    idx = []
    for g in range(_NP):
        for f in range(_NANG):
            if g < _NAA:
                idx += [2 * g + f if f < 2 else ZR,
                        B_OM + g if f == 2 else ZR,
                        sc_row(g, f - 3) if f >= 3 else ZR]
            else:
                idx += [ZR, ZR, ZR]
    for base in (B_B1, B_W2):
        for g in range(_NP):
            if g < _NAA:
                idx += [base + g, base + 20 + g,
                        base + 40 + pos[g] if g in pos else ZR]
            else:
                idx += [ZR, ZR, ZR]
    assert rows.shape[0] == 212
    picked = jnp.take(rows, jnp.asarray(idx, jnp.int32), axis=0)
    nw1 = _NP * _NANG * 3
    w1 = picked[:nw1].reshape(_NP, _NANG, 3 * _HID)
    b1 = picked[nw1:nw1 + _NP * 3].reshape(_NP, 3 * _HID)
    w2 = picked[nw1 + _NP * 3:].reshape(_NP, 3 * _HID)

    svec = jnp.concatenate(
        [kde_params[grp][str(i)]['b2']
         for grp in ('bb', 'omega', 'sc') for i in range(_NAA)
         if str(i) in kde_params[grp]]
        + [weight_bb, weight_omega, weight_sc,
           jnp.zeros((1,), jnp.float32)])
    SZ = 79
    sidx = []
    for g in range(_NP):
        sidx += ([g, 20 + g, 40 + pos[g] if g in pos else SZ]
                 if g < _NAA else [SZ, SZ, SZ])
    for g in range(_NP):
        sidx += [57, 58, 59 + g if g < _NAA else SZ]
    spicked = jnp.take(svec, jnp.asarray(sidx, jnp.int32))
    b2 = spicked[:_NP * 3].reshape(_NP, 3)
    wpk = spicked[_NP * 3:].reshape(_NP, 3)
    return w1, b1, w2, b2, wpk


def kernel(atom_description, angles, alternatives, weight_omega, weight_bb,
           weight_sc, kde_params):
    naltern = alternatives.shape[-1]
    assert naltern == _A and angles.shape == (_B, _C, _R, _A, _NANG)

    slab = jnp.transpose(angles[:, :, :_NP], (2, 4, 0, 1, 3))
    slab = slab.reshape(_NP, _NANG, _N)

    w1, b1, w2, b2, wpk = _pack_params(
        kde_params, weight_bb, weight_omega, weight_sc)

    full = lambda a: pl.BlockSpec(a.shape, lambda i: (0,) * a.ndim)
    ins = (slab, w1, b1, w2, b2, wpk)
    out_spec = pl.BlockSpec((_B, _C, _RB, _A), lambda i: (0, 0, i, 0))
    out_sd = jax.ShapeDtypeStruct((_B, _C, _R, _A), jnp.float32)
    bb_score, rot = pl.pallas_call(
        _fused_kernel,
        grid=(_R // _RB,),
        in_specs=[full(a) for a in ins],
        out_specs=(out_spec, out_spec),
        out_shape=(out_sd, out_sd),
        compiler_params=pltpu.CompilerParams(
            dimension_semantics=("arbitrary",)),
    )(*ins)
    return (bb_score, rot)
